# baseline (speedup 1.0000x reference)
import functools

import jax
import jax.numpy as jnp
from jax import lax
from jax.experimental import pallas as pl
from jax.experimental.pallas import tpu as pltpu
from jax.experimental.pallas import tpu_sc as plsc

NC = 2
NS = 16
K = 80


@functools.lru_cache(maxsize=None)
def _build_scatter(N, E, D):
    NW = NC * NS
    EPW = E // NW
    CH = EPW // K
    NP = ((N + NS * K - 1) // (NS * K)) * (NS * K)
    RPT = NP // NS
    assert EPW * NW == E and CH * K == EPW and RPT % K == 0 and CH % 4 == 1

    mesh = plsc.VectorSubcoreMesh(core_axis_name="c", subcore_axis_name="s")

    @functools.partial(
        pl.kernel,
        mesh=mesh,
        out_type=jax.ShapeDtypeStruct((NC, NP, D), jnp.float32),
        scratch_types=(
            [pltpu.VMEM((K,), jnp.int32)] * 4
            + [pltpu.VMEM((K,), jnp.int32)] * 4
            + [pltpu.VMEM((K, D), jnp.float32)] * 4
            + [
                pltpu.VMEM_SHARED((NP, D), jnp.float32),
                pltpu.SemaphoreType.DMA,
                pltpu.SemaphoreType.DMA,
                pltpu.SemaphoreType.DMA,
                pltpu.SemaphoreType.DMA,
            ]
        ),
    )
    def scatter_kernel(feat_hbm, edge_hbm, out_hbm,
                       si0, si1, si2, si3, di0, di1, di2, di3,
                       r0, r1, r2, r3, hpart, gsem, s_isem, d_isem, ssem):
        sidx = [si0, si1, si2, si3]
        didx = [di0, di1, di2, di3]
        rows = [r0, r1, r2, r3]
        cid = lax.axis_index("c")
        sid = lax.axis_index("s")
        wid = sid * NC + cid
        base0 = wid * EPW

        def zrow(r, carry):
            for cc in range(D // 16):
                r0[r, pl.ds(cc * 16, 16)] = jnp.zeros((16,), jnp.float32)
            return carry
        lax.fori_loop(0, K, zrow, 0)
        for qz in range(RPT // K):
            pltpu.sync_copy(r0, hpart.at[pl.ds(sid * RPT + qz * K, K)])

        def sidx_load(c, b):
            pltpu.async_copy(edge_hbm.at[pl.ds(base0 + c * K, K)],
                             sidx[b], s_isem)

        def didx_load(c, b):
            pltpu.async_copy(edge_hbm.at[pl.ds(E + base0 + c * K, K)],
                             didx[b], d_isem)

        def sidx_drain(b):
            pltpu.make_async_copy(edge_hbm.at[pl.ds(0, K)], sidx[b],
                                  s_isem).wait()

        def didx_drain(b):
            pltpu.make_async_copy(edge_hbm.at[pl.ds(0, K)], didx[b],
                                  d_isem).wait()

        def gather(b, rb):
            pltpu.async_copy(feat_hbm.at[sidx[b]], rows[rb], gsem)

        def gdrain(rb):
            pltpu.make_async_copy(feat_hbm.at[pl.ds(0, K)], rows[rb],
                                  gsem).wait()

        def sdrain():
            pltpu.make_async_copy(r0, hpart.at[pl.ds(0, K)], ssem).wait()

        for b in range(3):
            pltpu.sync_copy(edge_hbm.at[pl.ds(base0 + b * K, K)], sidx[b])
            gather(b, b)
        sidx_load(3, 3)
        didx_load(0, 0)
        didx_load(1, 1)
        plsc.subcore_barrier()

        def chunk_body(q, b, first):
            gdrain(b)
            didx_drain(b)
            if first:
                pl.when(q >= 1)(sdrain)
            else:
                sdrain()
            pltpu.async_copy(rows[b], hpart.at[didx[b]], ssem, add=True)
            sidx_load(jnp.minimum(q + 4, CH - 1), b)
            didx_load(jnp.minimum(q + 2, CH - 1), (b + 2) % 4)
            sidx_drain((b + 3) % 4)
            gather((b + 3) % 4, (b + 3) % 4)

        def group(p, carry):
            q0 = 4 * p
            for b in range(4):
                chunk_body(q0 + b, b, b == 0)
            return carry
        lax.fori_loop(0, CH // 4, group, 0)

        gdrain(0)
        didx_drain(0)
        sdrain()
        pltpu.sync_copy(r0, hpart.at[di0], add=True)
        gdrain(1)
        gdrain(2)
        sidx_drain(3)
        didx_drain(1)

        plsc.subcore_barrier()
        pltpu.sync_copy(hpart.at[pl.ds(sid * RPT, RPT)],
                        out_hbm.at[cid, pl.ds(sid * RPT, RPT)])

    return scatter_kernel


def _linear_body(p_ref, w_ref, b_ref, o_ref):
    x = p_ref[0] + p_ref[1]
    o_ref[...] = (
        lax.dot_general(x, w_ref[...], (((1,), (1,)), ((), ())),
                        preferred_element_type=jnp.float32)
        + b_ref[...]
    )


@functools.lru_cache(maxsize=None)
def _build_linear(N, NP, D, BM):
    grid = (N // BM,)
    return pl.pallas_call(
        _linear_body,
        grid=grid,
        in_specs=[
            pl.BlockSpec((NC, BM, D), lambda i: (0, i, 0)),
            pl.BlockSpec((D, D), lambda i: (0, 0)),
            pl.BlockSpec((1, D), lambda i: (0, 0)),
        ],
        out_specs=pl.BlockSpec((BM, D), lambda i: (i, 0)),
        out_shape=jax.ShapeDtypeStruct((N, D), jnp.float32),
    )


def kernel(feature, edge_index, W, b):
    N, D = feature.shape
    E = edge_index.shape[1]
    eflat = edge_index.reshape(2 * E)
    partials = _build_scatter(N, E, D)(feature, eflat)
    out = _build_linear(N, partials.shape[1], D, 10000)(
        partials, W, b.reshape(1, D).astype(jnp.float32))
    return out


# speedup vs baseline: 1.0011x; 1.0011x over previous
import functools

import jax
import jax.numpy as jnp
from jax import lax
from jax.experimental import pallas as pl
from jax.experimental.pallas import tpu as pltpu
from jax.experimental.pallas import tpu_sc as plsc

NC = 2
NS = 16
K = 80


@functools.lru_cache(maxsize=None)
def _build_scatter(N, E, D):
    NW = NC * NS
    EPW = E // NW
    CH = EPW // K
    NP = ((N + NS * K - 1) // (NS * K)) * (NS * K)
    RPT = NP // NS
    assert EPW * NW == E and CH * K == EPW and RPT % K == 0 and CH % 4 == 1

    mesh = plsc.VectorSubcoreMesh(core_axis_name="c", subcore_axis_name="s")

    @functools.partial(
        pl.kernel,
        mesh=mesh,
        out_type=jax.ShapeDtypeStruct((NC, NP, D), jnp.float32),
        scratch_types=(
            [pltpu.VMEM((K,), jnp.int32)] * 4
            + [pltpu.VMEM((K,), jnp.int32)] * 4
            + [pltpu.VMEM((K, D), jnp.float32)] * 4
            + [
                pltpu.VMEM_SHARED((NP, D), jnp.float32),
                pltpu.SemaphoreType.DMA,
                pltpu.SemaphoreType.DMA,
                pltpu.SemaphoreType.DMA,
                pltpu.SemaphoreType.DMA,
            ]
        ),
    )
    def scatter_kernel(feat_hbm, edge_hbm, out_hbm,
                       si0, si1, si2, si3, di0, di1, di2, di3,
                       r0, r1, r2, r3, hpart, gsem, s_isem, d_isem, ssem):
        sidx = [si0, si1, si2, si3]
        didx = [di0, di1, di2, di3]
        rows = [r0, r1, r2, r3]
        cid = lax.axis_index("c")
        sid = lax.axis_index("s")
        wid = sid * NC + cid
        base0 = wid * EPW

        def sidx_load(c, b):
            pltpu.async_copy(edge_hbm.at[pl.ds(base0 + c * K, K)],
                             sidx[b], s_isem)

        def didx_load(c, b):
            pltpu.async_copy(edge_hbm.at[pl.ds(E + base0 + c * K, K)],
                             didx[b], d_isem)

        def sidx_drain(b):
            pltpu.make_async_copy(edge_hbm.at[pl.ds(0, K)], sidx[b],
                                  s_isem).wait()

        def didx_drain(b):
            pltpu.make_async_copy(edge_hbm.at[pl.ds(0, K)], didx[b],
                                  d_isem).wait()

        def gather(b, rb):
            pltpu.async_copy(feat_hbm.at[sidx[b]], rows[rb], gsem)

        def gdrain(rb):
            pltpu.make_async_copy(feat_hbm.at[pl.ds(0, K)], rows[rb],
                                  gsem).wait()

        def sdrain():
            pltpu.make_async_copy(r0, hpart.at[pl.ds(0, K)], ssem).wait()

        for b in range(3):
            pltpu.sync_copy(edge_hbm.at[pl.ds(base0 + b * K, K)], sidx[b])
            gather(b, b)
        sidx_load(3, 3)
        didx_load(0, 0)
        didx_load(1, 1)

        def zrow(r, carry):
            for cc in range(D // 16):
                r3[r, pl.ds(cc * 16, 16)] = jnp.zeros((16,), jnp.float32)
            return carry
        lax.fori_loop(0, K, zrow, 0)
        for qz in range(RPT // K):
            pltpu.sync_copy(r3, hpart.at[pl.ds(sid * RPT + qz * K, K)])
        plsc.subcore_barrier()

        def chunk_body(q, b, first):
            gdrain(b)
            didx_drain(b)
            if first:
                pl.when(q >= 1)(sdrain)
            else:
                sdrain()
            pltpu.async_copy(rows[b], hpart.at[didx[b]], ssem, add=True)
            sidx_load(jnp.minimum(q + 4, CH - 1), b)
            didx_load(jnp.minimum(q + 2, CH - 1), (b + 2) % 4)
            sidx_drain((b + 3) % 4)
            gather((b + 3) % 4, (b + 3) % 4)

        def group(p, carry):
            q0 = 4 * p
            for b in range(4):
                chunk_body(q0 + b, b, b == 0)
            return carry
        lax.fori_loop(0, CH // 4, group, 0)

        gdrain(0)
        didx_drain(0)
        sdrain()
        pltpu.sync_copy(r0, hpart.at[di0], add=True)
        gdrain(1)
        gdrain(2)
        sidx_drain(3)
        didx_drain(1)

        plsc.subcore_barrier()
        pltpu.sync_copy(hpart.at[pl.ds(sid * RPT, RPT)],
                        out_hbm.at[cid, pl.ds(sid * RPT, RPT)])

    return scatter_kernel


def _linear_body(p_ref, w_ref, b_ref, o_ref):
    x = p_ref[0] + p_ref[1]
    o_ref[...] = (
        lax.dot_general(x, w_ref[...], (((1,), (1,)), ((), ())),
                        preferred_element_type=jnp.float32)
        + b_ref[...]
    )


@functools.lru_cache(maxsize=None)
def _build_linear(N, NP, D, BM):
    grid = (N // BM,)
    return pl.pallas_call(
        _linear_body,
        grid=grid,
        in_specs=[
            pl.BlockSpec((NC, BM, D), lambda i: (0, i, 0)),
            pl.BlockSpec((D, D), lambda i: (0, 0)),
            pl.BlockSpec((1, D), lambda i: (0, 0)),
        ],
        out_specs=pl.BlockSpec((BM, D), lambda i: (i, 0)),
        out_shape=jax.ShapeDtypeStruct((N, D), jnp.float32),
    )


def kernel(feature, edge_index, W, b):
    N, D = feature.shape
    E = edge_index.shape[1]
    eflat = edge_index.reshape(2 * E)
    partials = _build_scatter(N, E, D)(feature, eflat)
    out = _build_linear(N, partials.shape[1], D, 10000)(
        partials, W, b.reshape(1, D).astype(jnp.float32))
    return out
